# baseline (speedup 1.0000x reference)
import jax
from jax.experimental import pallas as pl
from jax.experimental.pallas import tpu as pltpu


def _copy_body(w_ref, p_ref, t_ref, wo_ref, po_ref, to_ref, s0, s1, s2):
    c0 = pltpu.make_async_copy(w_ref, wo_ref, s0)
    c1 = pltpu.make_async_copy(p_ref, po_ref, s1)
    c2 = pltpu.make_async_copy(t_ref, to_ref, s2)
    c0.start()
    c1.start()
    c2.start()
    c0.wait()
    c1.wait()
    c2.wait()


def kernel(word_idx_tensor, pos_idx_tensor, true_tree_heads):
    out_shape = tuple(
        jax.ShapeDtypeStruct(x.shape, x.dtype)
        for x in (word_idx_tensor, pos_idx_tensor, true_tree_heads)
    )
    any_spec = pl.BlockSpec(memory_space=pl.ANY)
    return pl.pallas_call(
        _copy_body,
        out_shape=out_shape,
        in_specs=[any_spec] * 3,
        out_specs=[any_spec] * 3,
        scratch_shapes=[pltpu.SemaphoreType.DMA] * 3,
    )(word_idx_tensor, pos_idx_tensor, true_tree_heads)
